# Initial kernel scaffold; baseline (speedup 1.0000x reference)
import jax
import jax.numpy as jnp
from jax.experimental import pallas as pl


def kernel(x_n0, x_index_n0, edge_index_n0_to_n0, W_lin, W_conv, b_conv, w_al, b_al, w_ar, b_ar):
    raise NotImplementedError("write your pallas kernel here")


# speedup vs baseline: 25.6257x; 25.6257x over previous
import functools

import jax
import jax.numpy as jnp
from jax import lax
from jax.experimental import pallas as pl
from jax.experimental.pallas import tpu as pltpu
from jax.experimental.pallas import tpu_sc as plsc

N = 10000
D = 128
HALF = 5120
NR = 6144
RPT = 384
EPT = 20480
CH = 128
NCH = EPT // CH
GDR = 80
DDR = NR // CH


def _prep_body(x_ref, wlin_ref, wconv_ref, bconv_ref, war_ref,
               pg_ref, hb1_ref, b0_ref, g_ref):
    x = x_ref[...]
    h = lax.dot_general(x, wlin_ref[...], (((1,), (1,)), ((), ())),
                        preferred_element_type=jnp.float32)
    sr = lax.dot_general(h, war_ref[...], (((1,), (1,)), ((), ())),
                         preferred_element_type=jnp.float32)
    g = jnp.exp(sr - jnp.max(sr))
    logits = lax.dot_general(x, wconv_ref[...], (((1,), (1,)), ((), ())),
                             preferred_element_type=jnp.float32)
    logits = logits + bconv_ref[...]
    m = jnp.max(logits, axis=1, keepdims=True)
    eb = jnp.exp(logits - m)
    beta = eb / jnp.sum(eb, axis=1, keepdims=True)
    pg_ref[...] = h * g
    hb1_ref[...] = h * beta[:, 1:2]
    b0_ref[...] = beta[:, 0:1]
    g_ref[...] = g


def _combine_body(parts_ref, hb1_ref, b0_ref, den_ref, out_ref):
    agg = jnp.concatenate(
        [parts_ref[0][:HALF, :], parts_ref[1][:N - HALF, :]], axis=0)
    scale = b0_ref[...] / (den_ref[...] + 1e-30)
    out_ref[...] = agg * scale + hb1_ref[...]


_edge_mesh = plsc.VectorSubcoreMesh(core_axis_name="c", subcore_axis_name="s")


def _den_update(den_ref, d16, gv):
    io16 = lax.broadcasted_iota(jnp.int32, (16,), 0)
    k, v = plsc.sort_key_val(d16, gv)
    csum = plsc.cumsum(v)
    kprev = jnp.take_along_axis(k, jnp.maximum(io16 - 1, 0), axis=0)
    starts = jnp.logical_or(k != kprev, io16 == 0)
    knext = jnp.take_along_axis(k, jnp.minimum(io16 + 1, 15), axis=0)
    ends = jnp.logical_or(k != knext, io16 == 15)
    sidx = plsc.cummax(jnp.where(starts, io16, 0))
    cz = jnp.where(io16 == 0, 0.0,
                   jnp.take_along_axis(csum, jnp.maximum(io16 - 1, 0), axis=0))
    pc = jnp.take_along_axis(cz, sidx, axis=0)
    plsc.addupdate_scatter(den_ref,
                           [lax.shift_right_logical(k, 7),
                            lax.bitwise_and(k, 127)], csum - pc, mask=ends)


@functools.partial(
    pl.kernel,
    mesh=_edge_mesh,
    compiler_params=pltpu.CompilerParams(needs_layout_passes=False),
    out_type=[jax.ShapeDtypeStruct((2, NR, D), jnp.float32),
              jax.ShapeDtypeStruct((2, DDR, CH), jnp.float32)],
    scratch_types=[
        pltpu.VMEM((NCH, CH), jnp.int32),
        pltpu.VMEM((NCH, CH), jnp.int32),
        pltpu.VMEM((CH, D), jnp.float32),
        pltpu.VMEM((GDR, CH), jnp.float32),
        pltpu.VMEM((DDR, CH), jnp.float32),
        pltpu.VMEM((1, DDR), jnp.int32),
        pltpu.VMEM_SHARED((NR, D), jnp.float32),
        pltpu.VMEM_SHARED((DDR, CH), jnp.float32),
        pltpu.SemaphoreType.DMA,
    ],
)
def _edge_kernel(pg_hbm, src_hbm, dstl_hbm, g_hbm, zrow_hbm, iden_hbm,
                 out_hbm, outden_hbm,
                 src_v, dst_v, rows_v, g_v, den_v, iden_v,
                 accum, den_sh, sem):
    c = lax.axis_index("c")
    s = lax.axis_index("s")

    pltpu.sync_copy(src_hbm.at[s], src_v)
    pltpu.sync_copy(dstl_hbm.at[c].at[s], dst_v)
    pltpu.sync_copy(g_hbm, g_v)
    pltpu.sync_copy(iden_hbm, iden_v)

    pltpu.sync_copy(zrow_hbm.at[pl.ds(0, DDR)], den_v)
    pltpu.sync_copy(zrow_hbm, rows_v)
    for k in range(RPT // CH):
        pltpu.sync_copy(rows_v, accum.at[pl.ds(s * RPT + k * CH, CH)])

    @pl.when(s == 0)
    def _():
        pltpu.sync_copy(den_v, den_sh)

    plsc.subcore_barrier()

    def row_body(j, carry):
        pltpu.async_copy(pg_hbm.at[src_v.at[j]], rows_v, sem).wait()
        pltpu.sync_copy(rows_v, accum.at[dst_v.at[j]], add=True)
        for t in range(CH // 16):
            s16 = src_v[j, pl.ds(t * 16, 16)]
            d16 = dst_v[j, pl.ds(t * 16, 16)]
            gv = plsc.load_gather(g_v, [lax.shift_right_logical(s16, 7),
                                        lax.bitwise_and(s16, 127)])
            _den_update(den_v, d16, gv)
        return carry

    lax.fori_loop(0, NCH, row_body, 0)

    pltpu.sync_copy(den_v, den_sh.at[iden_v.at[0]], add=True)
    plsc.subcore_barrier()

    for k in range(RPT // CH):
        r0 = s * RPT + k * CH
        pltpu.sync_copy(accum.at[pl.ds(r0, CH)], rows_v)
        pltpu.sync_copy(rows_v, out_hbm.at[c].at[pl.ds(r0, CH)])

    @pl.when(s == 0)
    def _():
        pltpu.sync_copy(den_sh, den_v)
        pltpu.sync_copy(den_v, outden_hbm.at[c])


def kernel(x_n0, x_index_n0, edge_index_n0_to_n0, W_lin, W_conv, b_conv,
           w_al, b_al, w_ar, b_ar):
    del x_index_n0, w_al, b_al, b_ar
    pg, hb1, b0, g = pl.pallas_call(
        _prep_body,
        out_shape=[jax.ShapeDtypeStruct((N, D), jnp.float32),
                   jax.ShapeDtypeStruct((N, D), jnp.float32),
                   jax.ShapeDtypeStruct((N, 1), jnp.float32),
                   jax.ShapeDtypeStruct((N, 1), jnp.float32)],
    )(x_n0, W_lin, W_conv, b_conv.reshape(1, 2), w_ar)

    dst = edge_index_n0_to_n0[0].astype(jnp.int32)
    src = edge_index_n0_to_n0[1].astype(jnp.int32)
    e = src.shape[0]
    epad = 16 * EPT
    npad = epad - e
    pad_iota = jnp.arange(npad, dtype=jnp.int32)
    src_f = jnp.concatenate([src, pad_iota % jnp.int32(N)])
    dst_f = jnp.concatenate(
        [dst, jnp.int32(N) + pad_iota % jnp.int32(2 * HALF - N)])
    spread = jnp.int32(HALF) + (dst_f % jnp.int32(NR - HALF))
    loc0 = jnp.where(dst_f < HALF, dst_f, spread)
    loc1 = jnp.where(dst_f >= HALF, dst_f - jnp.int32(HALF), spread)
    src_p = src_f.reshape(16, NCH, CH)
    dstl = jnp.stack([loc0, loc1]).reshape(2, 16, NCH, CH)
    gflat = jnp.concatenate(
        [g.reshape(N), jnp.zeros((GDR * CH - N,), jnp.float32)]
    ).reshape(GDR, CH)
    zrow = jnp.zeros((CH, CH), jnp.float32)
    iden = jnp.arange(DDR, dtype=jnp.int32).reshape(1, DDR)

    parts, denp = _edge_kernel(pg, src_p, dstl, gflat, zrow, iden)

    den = jnp.concatenate(
        [denp[0].reshape(NR)[:HALF], denp[1].reshape(NR)[:N - HALF]]
    ).reshape(N, 1)

    out = pl.pallas_call(
        _combine_body,
        out_shape=jax.ShapeDtypeStruct((N, D), jnp.float32),
    )(parts, hb1, b0, den)
    return out
